# baseline (speedup 1.0000x reference)
import functools

import jax
import jax.numpy as jnp
from jax import lax
from jax.experimental import pallas as pl
from jax.experimental.pallas import tpu as pltpu
from jax.experimental.pallas import tpu_sc as plsc

N_USERS_K = 10000
N_ITEMS_K = 40000
NN = N_USERS_K + N_ITEMS_K
EDGES = 800000
DD = 64
HALF = DD // 2
NLAYERS = 3
REG_K = 0.01
BATCH_K = 16384

NC = 2
NS = 16
E_PER_TILE = EDGES // NS
BLK = 16
NB = 25
CHUNK = BLK * NB
N_CHUNKS = E_PER_TILE // CHUNK
RING_R = 2
RING_V = 4
ZROWS = 400
TOTAL_SLABS = NN // ZROWS

_sc_mesh = plsc.VectorSubcoreMesh(core_axis_name="c", subcore_axis_name="s")


@functools.partial(
    pl.kernel,
    mesh=_sc_mesh,
    compiler_params=pltpu.CompilerParams(use_tc_tiling_on_sc=False),
    out_type=jax.ShapeDtypeStruct((2 * NN, HALF), jnp.float32),
    scratch_types=[
        pltpu.VMEM((RING_R * CHUNK,), jnp.int32),
        pltpu.VMEM((RING_V * NB, BLK), jnp.int32),
        pltpu.VMEM((RING_R * CHUNK,), jnp.float32),
        pltpu.VMEM((RING_R * CHUNK, HALF), jnp.float32),
        pltpu.VMEM_SHARED((NN, HALF), jnp.float32),
        pltpu.SemaphoreType.DMA,
        pltpu.SemaphoreType.DMA,
        pltpu.SemaphoreType.DMA,
    ],
)
def _sc_spmm(ego_flat, erow, ecol, eval_, out, colv, rowv, valv, rows,
             acc, sem_i, sem_g, sem_sc):
    c = lax.axis_index("c")
    s = lax.axis_index("s")
    c_off = c * NN

    zero16 = jnp.zeros((16,), jnp.float32)
    zstg = rows.at[pl.ds(0, ZROWS)]

    def _zrow(r, _):
        rows[r, pl.ds(0, 16)] = zero16
        rows[r, pl.ds(16, 16)] = zero16
        return 0

    lax.fori_loop(0, ZROWS, _zrow, 0)
    n_my = (TOTAL_SLABS - s + NS - 1) // NS

    def _zslab(z, _):
        r0 = (s + z * NS) * ZROWS
        pltpu.sync_copy(zstg, acc.at[pl.ds(r0, ZROWS)])
        return 0

    lax.fori_loop(0, n_my, _zslab, 0)
    plsc.subcore_barrier()

    def _idx_refs(t):
        bufr = lax.rem(t, RING_R)
        bufv = lax.rem(t, RING_V)
        eb = s * E_PER_TILE + t * CHUNK
        vb = bufr * CHUNK
        rb = s * (E_PER_TILE // BLK) + t * NB
        return [(ecol.at[pl.ds(eb, CHUNK)], colv.at[pl.ds(vb, CHUNK)]),
                (eval_.at[pl.ds(eb, CHUNK)], valv.at[pl.ds(vb, CHUNK)]),
                (erow.at[pl.ds(rb, NB)], rowv.at[pl.ds(bufv * NB, NB)])]

    def _fire_idx(t):
        for src, dst in _idx_refs(t):
            pltpu.async_copy(src, dst, sem_i)

    def _drain_idx(t):
        for src, dst in _idx_refs(t):
            pltpu.make_async_copy(src, dst, sem_i).wait()

    def _g_refs(t):
        vb = lax.rem(t, RING_R) * CHUNK
        ego_c = ego_flat.at[pl.ds(c * NN, NN)]
        return [(ego_c.at[colv.at[pl.ds(vb + q * BLK, BLK)]],
                 rows.at[pl.ds(vb + q * BLK, BLK)]) for q in range(NB)]

    def _fire_g(t):
        for src, dst in _g_refs(t):
            pltpu.async_copy(src, dst, sem_g)

    def _drain_g(t):
        for src, dst in _g_refs(t):
            pltpu.make_async_copy(src, dst, sem_g).wait()

    def _scale(t):
        vb = lax.rem(t, RING_R) * CHUNK

        def _go(g, _):
            o = vb + g * 16
            vv = valv[pl.ds(o, 16)]
            for el in range(16):
                e = o + el
                m = vv[el]
                rows[e, pl.ds(0, 16)] = rows[e, pl.ds(0, 16)] * m
                rows[e, pl.ds(16, 16)] = rows[e, pl.ds(16, 16)] * m
            return 0

        lax.fori_loop(0, CHUNK // 16, _go, 0)

    def _sc_refs(t):
        bufr = lax.rem(t, RING_R)
        bufv = lax.rem(t, RING_V)
        vb = bufr * CHUNK
        return [(rows.at[pl.ds(vb + q * BLK, BLK)],
                 acc.at[rowv.at[bufv * NB + q]]) for q in range(NB)]

    def _fire_sc(t):
        for src, dst in _sc_refs(t):
            pltpu.async_copy(src, dst, sem_sc, add=True)

    def _drain_sc(t):
        for src, dst in _sc_refs(t):
            pltpu.make_async_copy(src, dst, sem_sc).wait()

    _fire_idx(0)
    _drain_idx(0)
    _fire_g(0)
    _fire_idx(1)

    _drain_idx(1)
    _drain_g(0)
    _fire_g(1)
    _scale(0)
    _fire_sc(0)
    _fire_idx(2)

    def _steady(t, _):
        _drain_idx(t + 1)
        _drain_g(t)
        _drain_sc(t - 1)
        _fire_g(t + 1)
        _scale(t)
        _fire_sc(t)
        _fire_idx(t + 2)
        return 0

    lax.fori_loop(1, N_CHUNKS - 2, _steady, 0)

    t = N_CHUNKS - 2
    _drain_idx(t + 1)
    _drain_g(t)
    _drain_sc(t - 1)
    _fire_g(t + 1)
    _scale(t)
    _fire_sc(t)
    t = N_CHUNKS - 1
    _drain_g(t)
    _scale(t)
    _drain_sc(t - 1)
    _fire_sc(t)
    _drain_sc(t)
    plsc.subcore_barrier()

    def _dslab(z, _):
        r0 = (s + z * NS) * ZROWS
        pltpu.sync_copy(acc.at[pl.ds(r0, ZROWS)], zstg)
        pltpu.sync_copy(zstg, out.at[pl.ds(c_off + r0, ZROWS)])
        return 0

    lax.fori_loop(0, n_my, _dslab, 0)


_B_PER_W = BATCH_K // (NC * NS)
_GCHUNK = 128
_GQ = _B_PER_W // _GCHUNK


@functools.partial(
    pl.kernel,
    mesh=_sc_mesh,
    compiler_params=pltpu.CompilerParams(use_tc_tiling_on_sc=False),
    out_type=jax.ShapeDtypeStruct((3, 4, BATCH_K, DD), jnp.float32),
    scratch_types=[
        pltpu.VMEM((3, _GCHUNK), jnp.int32),
        pltpu.VMEM((12, _GCHUNK, DD), jnp.float32),
        pltpu.SemaphoreType.DMA,
        pltpu.SemaphoreType.DMA,
    ],
)
def _sc_gather(t0, t1, t2, t3, iu, ii, ij, out, bidx, grows, sem_i, sem_g):
    c = lax.axis_index("c")
    s = lax.axis_index("s")
    w = s * NC + c

    def _q(q, _):
        b0 = w * _B_PER_W + q * _GCHUNK
        ips = [pltpu.async_copy(idxref.at[pl.ds(b0, _GCHUNK)], bidx.at[si],
                                sem_i)
               for si, idxref in enumerate((iu, ii, ij))]
        for cp in ips:
            cp.wait()
        gps = []
        for si in range(3):
            for t, tab in enumerate((t0, t1, t2, t3)):
                gps.append(pltpu.async_copy(tab.at[bidx.at[si]],
                                            grows.at[si * 4 + t], sem_g))
        for cp in gps:
            cp.wait()
        wps = []
        for si in range(3):
            for t in range(4):
                wps.append(pltpu.async_copy(
                    grows.at[si * 4 + t], out.at[si, t, pl.ds(b0, _GCHUNK)],
                    sem_i))
        for cp in wps:
            cp.wait()
        return 0

    lax.fori_loop(0, _GQ, _q, 0)


_RBLK = 5000
_NBLK = NN // _RBLK


def _tc_layer_body(ws_ref, eg_ref, w1_ref, b1_ref, w2_ref, b2_ref,
                   ego_out, norm_out):
    ws = jnp.concatenate([ws_ref[0], ws_ref[1]], axis=1)
    eg = jnp.concatenate([eg_ref[0], eg_ref[1]], axis=1)
    aff = eg * ws
    t = (jnp.dot(ws, w1_ref[...], preferred_element_type=jnp.float32)
         + b1_ref[...]
         + jnp.dot(aff, w2_ref[...], preferred_element_type=jnp.float32)
         + b2_ref[...])
    e2 = jnp.where(t >= 0, t, 0.01 * t)
    nr = jnp.sqrt(jnp.sum(e2 * e2, axis=1, keepdims=True))
    norm_out[...] = e2 / jnp.maximum(nr, 1e-12)
    ego_out[0] = e2[:, :HALF]
    ego_out[1] = e2[:, HALF:]


_tc_layer = pl.pallas_call(
    _tc_layer_body,
    grid=(_NBLK,),
    in_specs=[
        pl.BlockSpec((2, _RBLK, HALF), lambda i: (0, i, 0)),
        pl.BlockSpec((2, _RBLK, HALF), lambda i: (0, i, 0)),
        pl.BlockSpec((DD, DD), lambda i: (0, 0)),
        pl.BlockSpec((1, DD), lambda i: (0, 0)),
        pl.BlockSpec((DD, DD), lambda i: (0, 0)),
        pl.BlockSpec((1, DD), lambda i: (0, 0)),
    ],
    out_specs=[
        pl.BlockSpec((2, _RBLK, HALF), lambda i: (0, i, 0)),
        pl.BlockSpec((_RBLK, DD), lambda i: (i, 0)),
    ],
    out_shape=[
        jax.ShapeDtypeStruct((2, NN, HALF), jnp.float32),
        jax.ShapeDtypeStruct((NN, DD), jnp.float32),
    ],
)


_LBLK = 4096
_LGRID = BATCH_K // _LBLK


def _tc_loss_body(g_ref, out_ref, acc_ref):
    pid = pl.program_id(0)

    @pl.when(pid == 0)
    def _init():
        acc_ref[0] = 0.0
        acc_ref[1] = 0.0

    ue = g_ref[0]
    pe = g_ref[1]
    ne = g_ref[2]
    yui = jnp.sum(ue * pe, axis=(0, 2))
    yuj = jnp.sum(ue * ne, axis=(0, 2))
    sl = jnp.sum(jnp.log(jax.nn.sigmoid(yui - yuj)))
    sq = jnp.sum(ue * ue) + jnp.sum(pe * pe) + jnp.sum(ne * ne)
    acc_ref[0] = acc_ref[0] + sl
    acc_ref[1] = acc_ref[1] + sq

    @pl.when(pid == _LGRID - 1)
    def _fin():
        out_ref[0, 0] = (-acc_ref[0] / BATCH_K
                         + REG_K * (acc_ref[1] * 0.5 / BATCH_K))


_tc_loss = pl.pallas_call(
    _tc_loss_body,
    grid=(_LGRID,),
    in_specs=[pl.BlockSpec((3, 4, _LBLK, DD), lambda b: (0, 0, b, 0))],
    out_specs=pl.BlockSpec(memory_space=pltpu.SMEM),
    out_shape=jax.ShapeDtypeStruct((1, 1), jnp.float32),
    scratch_shapes=[pltpu.SMEM((2,), jnp.float32)],
)


def kernel(u, i, j, edge_row, edge_col, edge_val, u_embeddings, i_embeddings,
           W1, b1, W2, b2):
    ego0 = jnp.concatenate([u_embeddings, i_embeddings], axis=0)
    ego_h = jnp.stack([ego0[:, :HALF], ego0[:, HALF:]], axis=0)
    erow2 = edge_row.reshape(EDGES // BLK, BLK)
    norms = []
    for k in range(NLAYERS):
        ws_flat = _sc_spmm(ego_h.reshape(2 * NN, HALF), erow2,
                           edge_col, edge_val)
        ego_h, nrm = _tc_layer(ws_flat.reshape(2, NN, HALF), ego_h,
                               W1[k], b1[k].reshape(1, DD),
                               W2[k], b2[k].reshape(1, DD))
        norms.append(nrm)
    iu = u.astype(jnp.int32)
    ii = i.astype(jnp.int32) + N_USERS_K
    ij = j.astype(jnp.int32) + N_USERS_K
    g = _sc_gather(ego0, norms[0], norms[1], norms[2], iu, ii, ij)
    loss = _tc_loss(g)
    return loss[0, 0]


# speedup vs baseline: 1.0642x; 1.0642x over previous
import functools

import jax
import jax.numpy as jnp
from jax import lax
from jax.experimental import pallas as pl
from jax.experimental.pallas import tpu as pltpu
from jax.experimental.pallas import tpu_sc as plsc

N_USERS_K = 10000
N_ITEMS_K = 40000
NN = N_USERS_K + N_ITEMS_K
EDGES = 800000
DD = 64
HALF = DD // 2
NLAYERS = 3
REG_K = 0.01
BATCH_K = 16384

NC = 2
NS = 16
E_PER_TILE = EDGES // NS
BLK = 40
NB = 10
CHUNK = BLK * NB
N_CHUNKS = E_PER_TILE // CHUNK
RING_R = 2
RING_V = 4
ZROWS = 400
TOTAL_SLABS = NN // ZROWS

_sc_mesh = plsc.VectorSubcoreMesh(core_axis_name="c", subcore_axis_name="s")


@functools.partial(
    pl.kernel,
    mesh=_sc_mesh,
    compiler_params=pltpu.CompilerParams(use_tc_tiling_on_sc=False),
    out_type=jax.ShapeDtypeStruct((2 * NN, HALF), jnp.float32),
    scratch_types=[
        pltpu.VMEM((RING_R * CHUNK,), jnp.int32),
        pltpu.VMEM((RING_V * NB, BLK), jnp.int32),
        pltpu.VMEM((RING_R * CHUNK,), jnp.float32),
        pltpu.VMEM((RING_R * CHUNK, HALF), jnp.float32),
        pltpu.VMEM_SHARED((NN, HALF), jnp.float32),
        pltpu.SemaphoreType.DMA,
        pltpu.SemaphoreType.DMA,
        pltpu.SemaphoreType.DMA,
        pltpu.SemaphoreType.DMA,
    ],
)
def _sc_spmm(ego_flat, erow, ecol, eval_, out, colv, rowv, valv, rows,
             acc, sem_i, sem_g0, sem_g1, sem_sc):
    c = lax.axis_index("c")
    s = lax.axis_index("s")
    c_off = c * NN

    zero16 = jnp.zeros((16,), jnp.float32)
    zstg = rows.at[pl.ds(0, ZROWS)]

    def _zrow(r, _):
        rows[r, pl.ds(0, 16)] = zero16
        rows[r, pl.ds(16, 16)] = zero16
        return 0

    lax.fori_loop(0, ZROWS, _zrow, 0)
    n_my = (TOTAL_SLABS - s + NS - 1) // NS

    def _zslab(z, _):
        r0 = (s + z * NS) * ZROWS
        pltpu.sync_copy(zstg, acc.at[pl.ds(r0, ZROWS)])
        return 0

    lax.fori_loop(0, n_my, _zslab, 0)
    plsc.subcore_barrier()

    def _idx_refs(t):
        bufr = lax.rem(t, RING_R)
        bufv = lax.rem(t, RING_V)
        eb = s * E_PER_TILE + t * CHUNK
        vb = bufr * CHUNK
        rb = s * (E_PER_TILE // BLK) + t * NB
        return [(ecol.at[pl.ds(eb, CHUNK)], colv.at[pl.ds(vb, CHUNK)]),
                (eval_.at[pl.ds(eb, CHUNK)], valv.at[pl.ds(vb, CHUNK)]),
                (erow.at[pl.ds(rb, NB)], rowv.at[pl.ds(bufv * NB, NB)])]

    def _fire_idx(t):
        for src, dst in _idx_refs(t):
            pltpu.async_copy(src, dst, sem_i)

    def _drain_idx(t):
        for src, dst in _idx_refs(t):
            pltpu.make_async_copy(src, dst, sem_i).wait()

    def _g_refs(t):
        vb = lax.rem(t, RING_R) * CHUNK
        ego_c = ego_flat.at[pl.ds(c * NN, NN)]
        return [(ego_c.at[colv.at[pl.ds(vb + q * BLK, BLK)]],
                 rows.at[pl.ds(vb + q * BLK, BLK)]) for q in range(NB)]

    def _fire_g(t, sg):
        for src, dst in _g_refs(t):
            pltpu.async_copy(src, dst, sg)

    def _drain_g(t, sg):
        for src, dst in _g_refs(t):
            pltpu.make_async_copy(src, dst, sg).wait()

    def _scale(t):
        vb = lax.rem(t, RING_R) * CHUNK

        def _go(g, _):
            o = vb + g * 16
            vv = valv[pl.ds(o, 16)]
            for el in range(16):
                e = o + el
                m = vv[el]
                rows[e, pl.ds(0, 16)] = rows[e, pl.ds(0, 16)] * m
                rows[e, pl.ds(16, 16)] = rows[e, pl.ds(16, 16)] * m
            return 0

        lax.fori_loop(0, CHUNK // 16, _go, 0)

    def _sc_refs(t):
        bufr = lax.rem(t, RING_R)
        bufv = lax.rem(t, RING_V)
        vb = bufr * CHUNK
        return [(rows.at[pl.ds(vb + q * BLK, BLK)],
                 acc.at[rowv.at[bufv * NB + q]]) for q in range(NB)]

    def _fire_sc(t):
        for src, dst in _sc_refs(t):
            pltpu.async_copy(src, dst, sem_sc, add=True)

    def _drain_sc(t):
        for src, dst in _sc_refs(t):
            pltpu.make_async_copy(src, dst, sem_sc).wait()

    _fire_idx(0)
    _drain_idx(0)
    _fire_g(0, sem_g0)
    _fire_idx(1)

    _drain_idx(1)
    _fire_g(1, sem_g1)
    _drain_g(0, sem_g0)
    _scale(0)
    _fire_sc(0)
    _fire_idx(2)

    def _pair(p, _):
        t1 = 2 * p + 1
        _drain_idx(t1 + 1)
        _drain_sc(t1 - 1)
        _fire_g(t1 + 1, sem_g0)
        _drain_g(t1, sem_g1)
        _scale(t1)
        _fire_sc(t1)
        _fire_idx(t1 + 2)
        t2 = t1 + 1
        _drain_idx(t2 + 1)
        _drain_sc(t2 - 1)
        _fire_g(t2 + 1, sem_g1)
        _drain_g(t2, sem_g0)
        _scale(t2)
        _fire_sc(t2)
        _fire_idx(t2 + 2)
        return 0

    lax.fori_loop(0, (N_CHUNKS - 5) // 2, _pair, 0)

    t = N_CHUNKS - 4
    _drain_idx(t + 1)
    _drain_sc(t - 1)
    _fire_g(t + 1, sem_g0)
    _drain_g(t, sem_g1)
    _scale(t)
    _fire_sc(t)
    _fire_idx(t + 2)
    t = N_CHUNKS - 3
    _drain_idx(t + 1)
    _drain_sc(t - 1)
    _fire_g(t + 1, sem_g1)
    _drain_g(t, sem_g0)
    _scale(t)
    _fire_sc(t)
    _fire_idx(t + 2)
    t = N_CHUNKS - 2
    _drain_idx(t + 1)
    _drain_sc(t - 1)
    _fire_g(t + 1, sem_g0)
    _drain_g(t, sem_g1)
    _scale(t)
    _fire_sc(t)
    t = N_CHUNKS - 1
    _drain_sc(t - 1)
    _drain_g(t, sem_g0)
    _scale(t)
    _fire_sc(t)
    _drain_sc(t)
    plsc.subcore_barrier()

    def _dslab(z, _):
        r0 = (s + z * NS) * ZROWS
        pltpu.sync_copy(acc.at[pl.ds(r0, ZROWS)], zstg)
        pltpu.sync_copy(zstg, out.at[pl.ds(c_off + r0, ZROWS)])
        return 0

    lax.fori_loop(0, n_my, _dslab, 0)


_B_PER_W = BATCH_K // (NC * NS)
_GCHUNK = 128
_GQ = _B_PER_W // _GCHUNK


@functools.partial(
    pl.kernel,
    mesh=_sc_mesh,
    compiler_params=pltpu.CompilerParams(use_tc_tiling_on_sc=False),
    out_type=jax.ShapeDtypeStruct((3, 4, BATCH_K, DD), jnp.float32),
    scratch_types=[
        pltpu.VMEM((3, _GCHUNK), jnp.int32),
        pltpu.VMEM((12, _GCHUNK, DD), jnp.float32),
        pltpu.SemaphoreType.DMA,
        pltpu.SemaphoreType.DMA,
    ],
)
def _sc_gather(t0, t1, t2, t3, iu, ii, ij, out, bidx, grows, sem_i, sem_g):
    c = lax.axis_index("c")
    s = lax.axis_index("s")
    w = s * NC + c

    def _q(q, _):
        b0 = w * _B_PER_W + q * _GCHUNK
        ips = [pltpu.async_copy(idxref.at[pl.ds(b0, _GCHUNK)], bidx.at[si],
                                sem_i)
               for si, idxref in enumerate((iu, ii, ij))]
        for cp in ips:
            cp.wait()
        gps = []
        for si in range(3):
            for t, tab in enumerate((t0, t1, t2, t3)):
                gps.append(pltpu.async_copy(tab.at[bidx.at[si]],
                                            grows.at[si * 4 + t], sem_g))
        for cp in gps:
            cp.wait()
        wps = []
        for si in range(3):
            for t in range(4):
                wps.append(pltpu.async_copy(
                    grows.at[si * 4 + t], out.at[si, t, pl.ds(b0, _GCHUNK)],
                    sem_i))
        for cp in wps:
            cp.wait()
        return 0

    lax.fori_loop(0, _GQ, _q, 0)


_RBLK = 5000
_NBLK = NN // _RBLK


def _tc_layer_body(ws_ref, eg_ref, w1_ref, b1_ref, w2_ref, b2_ref,
                   ego_out, norm_out):
    ws = jnp.concatenate([ws_ref[0], ws_ref[1]], axis=1)
    eg = jnp.concatenate([eg_ref[0], eg_ref[1]], axis=1)
    aff = eg * ws
    t = (jnp.dot(ws, w1_ref[...], preferred_element_type=jnp.float32)
         + b1_ref[...]
         + jnp.dot(aff, w2_ref[...], preferred_element_type=jnp.float32)
         + b2_ref[...])
    e2 = jnp.where(t >= 0, t, 0.01 * t)
    nr = jnp.sqrt(jnp.sum(e2 * e2, axis=1, keepdims=True))
    norm_out[...] = e2 / jnp.maximum(nr, 1e-12)
    ego_out[0] = e2[:, :HALF]
    ego_out[1] = e2[:, HALF:]


_tc_layer = pl.pallas_call(
    _tc_layer_body,
    grid=(_NBLK,),
    in_specs=[
        pl.BlockSpec((2, _RBLK, HALF), lambda i: (0, i, 0)),
        pl.BlockSpec((2, _RBLK, HALF), lambda i: (0, i, 0)),
        pl.BlockSpec((DD, DD), lambda i: (0, 0)),
        pl.BlockSpec((1, DD), lambda i: (0, 0)),
        pl.BlockSpec((DD, DD), lambda i: (0, 0)),
        pl.BlockSpec((1, DD), lambda i: (0, 0)),
    ],
    out_specs=[
        pl.BlockSpec((2, _RBLK, HALF), lambda i: (0, i, 0)),
        pl.BlockSpec((_RBLK, DD), lambda i: (i, 0)),
    ],
    out_shape=[
        jax.ShapeDtypeStruct((2, NN, HALF), jnp.float32),
        jax.ShapeDtypeStruct((NN, DD), jnp.float32),
    ],
)


_LBLK = 4096
_LGRID = BATCH_K // _LBLK


def _tc_loss_body(g_ref, out_ref, acc_ref):
    pid = pl.program_id(0)

    @pl.when(pid == 0)
    def _init():
        acc_ref[0] = 0.0
        acc_ref[1] = 0.0

    ue = g_ref[0]
    pe = g_ref[1]
    ne = g_ref[2]
    yui = jnp.sum(ue * pe, axis=(0, 2))
    yuj = jnp.sum(ue * ne, axis=(0, 2))
    sl = jnp.sum(jnp.log(jax.nn.sigmoid(yui - yuj)))
    sq = jnp.sum(ue * ue) + jnp.sum(pe * pe) + jnp.sum(ne * ne)
    acc_ref[0] = acc_ref[0] + sl
    acc_ref[1] = acc_ref[1] + sq

    @pl.when(pid == _LGRID - 1)
    def _fin():
        out_ref[0, 0] = (-acc_ref[0] / BATCH_K
                         + REG_K * (acc_ref[1] * 0.5 / BATCH_K))


_tc_loss = pl.pallas_call(
    _tc_loss_body,
    grid=(_LGRID,),
    in_specs=[pl.BlockSpec((3, 4, _LBLK, DD), lambda b: (0, 0, b, 0))],
    out_specs=pl.BlockSpec(memory_space=pltpu.SMEM),
    out_shape=jax.ShapeDtypeStruct((1, 1), jnp.float32),
    scratch_shapes=[pltpu.SMEM((2,), jnp.float32)],
)


def kernel(u, i, j, edge_row, edge_col, edge_val, u_embeddings, i_embeddings,
           W1, b1, W2, b2):
    ego0 = jnp.concatenate([u_embeddings, i_embeddings], axis=0)
    ego_h = jnp.stack([ego0[:, :HALF], ego0[:, HALF:]], axis=0)
    erow2 = edge_row.reshape(EDGES // BLK, BLK)
    norms = []
    for k in range(NLAYERS):
        ws_flat = _sc_spmm(ego_h.reshape(2 * NN, HALF), erow2,
                           edge_col, edge_val)
        ego_h, nrm = _tc_layer(ws_flat.reshape(2, NN, HALF), ego_h,
                               W1[k], b1[k].reshape(1, DD),
                               W2[k], b2[k].reshape(1, DD))
        norms.append(nrm)
    iu = u.astype(jnp.int32)
    ii = i.astype(jnp.int32) + N_USERS_K
    ij = j.astype(jnp.int32) + N_USERS_K
    g = _sc_gather(ego0, norms[0], norms[1], norms[2], iu, ii, ij)
    loss = _tc_loss(g)
    return loss[0, 0]
